# baseline (speedup 1.0000x reference)
import functools

import jax
import jax.numpy as jnp
from jax import lax
from jax.experimental import pallas as pl
from jax.experimental.pallas import tpu as pltpu
from jax.experimental.pallas import tpu_sc as plsc

NUM_NODES = 100000
MEM_DIM = 128
MSG_DIM = 128
PERIOD = 7
B = 16384
D_IN = MSG_DIM + MEM_DIM

NC = 2
NS = 16
L = 16
NW = NC * NS
BPW = B // NW

CP_CHUNK = 104
CP_STEPS = 32
CP_ROWS = CP_CHUNK * CP_STEPS
assert 31 * CP_ROWS >= NUM_NODES

_mesh = plsc.VectorSubcoreMesh(
    core_axis_name="c", subcore_axis_name="s", num_cores=NC, num_subcores=NS)


def _worker_id():
  return lax.axis_index("s") * NC + lax.axis_index("c")


@functools.partial(
    pl.kernel,
    out_type=jax.ShapeDtypeStruct((B, MEM_DIM), jnp.float32),
    mesh=_mesh,
    compiler_params=pltpu.CompilerParams(needs_layout_passes=False),
    scratch_types=[
        pltpu.VMEM((BPW,), jnp.int32),
        pltpu.VMEM((BPW, MEM_DIM), jnp.float32),
        pltpu.SemaphoreType.DMA,
    ],
)
def _gather_rows(table_hbm, idx_hbm, out_hbm, idx_v, rows_v, sem):
  base = _worker_id() * BPW
  pltpu.sync_copy(idx_hbm.at[pl.ds(base, BPW)], idx_v)
  pltpu.async_copy(table_hbm.at[idx_v], rows_v, sem).wait()
  pltpu.sync_copy(rows_v, out_hbm.at[pl.ds(base, BPW)])


def _compute_body(msg_ref, old_ref, wm_ref, wl_ref, bl_ref, g_ref, bt_ref,
                  out_ref):
  x1 = msg_ref[...] * wm_ref[:, :MSG_DIM]
  x2 = old_ref[...] * wm_ref[:, MSG_DIM:]
  dn = (((1,), (1,)), ((), ()))
  acc = lax.dot_general(x1, wl_ref[:, :MSG_DIM], dn,
                        preferred_element_type=jnp.float32,
                        precision=lax.Precision.HIGHEST)
  acc = acc + lax.dot_general(x2, wl_ref[:, MSG_DIM:], dn,
                              preferred_element_type=jnp.float32,
                              precision=lax.Precision.HIGHEST)
  acc = acc + bl_ref[...]
  mean = jnp.mean(acc, axis=-1, keepdims=True)
  var = jnp.mean((acc - mean) ** 2, axis=-1, keepdims=True)
  out_ref[...] = (acc - mean) / jnp.sqrt(var + 1e-5) * g_ref[...] + bt_ref[...]


_BLK = 1024
_compute = pl.pallas_call(
    _compute_body,
    grid=(B // _BLK,),
    in_specs=[
        pl.BlockSpec((_BLK, MSG_DIM), lambda i: (i, 0)),
        pl.BlockSpec((_BLK, MEM_DIM), lambda i: (i, 0)),
        pl.BlockSpec((1, D_IN), lambda i: (0, 0)),
        pl.BlockSpec((MEM_DIM, D_IN), lambda i: (0, 0)),
        pl.BlockSpec((1, MEM_DIM), lambda i: (0, 0)),
        pl.BlockSpec((1, MEM_DIM), lambda i: (0, 0)),
        pl.BlockSpec((1, MEM_DIM), lambda i: (0, 0)),
    ],
    out_specs=pl.BlockSpec((_BLK, MEM_DIM), lambda i: (i, 0)),
    out_shape=jax.ShapeDtypeStruct((B, MEM_DIM), jnp.float32),
)


@functools.partial(
    pl.kernel,
    out_type=jax.ShapeDtypeStruct((NUM_NODES,), jnp.int32),
    mesh=_mesh,
    compiler_params=pltpu.CompilerParams(needs_layout_passes=False),
    scratch_types=[
        pltpu.VMEM((NUM_NODES,), jnp.int32),
        pltpu.VMEM((B,), jnp.int32),
    ],
)
def _winner(idx_hbm, wtab_hbm, winner_v, allids):
  wid = _worker_id()
  iota = lax.iota(jnp.int32, L)

  @pl.when(wid == 0)
  def _build():
    pltpu.sync_copy(idx_hbm, allids)

    def build_outer(g):
      for j in range(16):
        o = g * 256 + j * L
        idv = allids[pl.ds(o, L)]
        _, is_last = plsc.scan_count(idv)
        plsc.store_scatter(winner_v, [idv], o + iota, mask=is_last)

    lax.fori_loop(0, B // 256, lambda g, _: (build_outer(g), 0)[1], 0)
    pltpu.sync_copy(winner_v, wtab_hbm)


def _copy_body(src_ref, dst_ref):
  dst_ref[...] = src_ref[...]


_CPBLK = 4000
_copy_tc = pl.pallas_call(
    _copy_body,
    grid=(NUM_NODES // _CPBLK,),
    in_specs=[pl.BlockSpec((_CPBLK, MEM_DIM), lambda i: (i, 0))],
    out_specs=pl.BlockSpec((_CPBLK, MEM_DIM), lambda i: (i, 0)),
    out_shape=jax.ShapeDtypeStruct((NUM_NODES, MEM_DIM), jnp.float32),
)


_SUB = 128

@functools.partial(
    pl.kernel,
    out_type=(),
    mesh=_mesh,
    compiler_params=pltpu.CompilerParams(needs_layout_passes=False),
    scratch_types=[
        pltpu.VMEM((BPW,), jnp.int32),
        pltpu.VMEM((BPW,), jnp.int32),
        pltpu.VMEM((BPW // _SUB, _SUB), jnp.int32),
        pltpu.VMEM((BPW, MEM_DIM), jnp.float32),
        pltpu.SemaphoreType.DMA,
        pltpu.SemaphoreType.DMA,
    ],
)
def _scatter_rows(idx_hbm, wtab_hbm, rows_hbm, out_hbm,
                  widxb, idsb, sidx, rbuf, sem, sem2):
  base = _worker_id() * BPW
  pltpu.sync_copy(idx_hbm.at[pl.ds(base, BPW)], idsb)
  pltpu.async_copy(wtab_hbm.at[idsb], widxb, sem).wait()
  gat = pltpu.async_copy(rows_hbm.at[widxb], rbuf, sem)
  for k in range(BPW // L):
    sidx[k * L // _SUB, pl.ds((k * L) % _SUB, L)] = idsb[pl.ds(k * L, L)]
  gat.wait()
  copies = [
      pltpu.async_copy(rbuf.at[pl.ds(s * _SUB, _SUB)],
                       out_hbm.at[sidx.at[s]], sem2)
      for s in range(BPW // _SUB)
  ]
  for c in copies:
    c.wait()


def kernel(node_ids, messages, node_memories, W_conv, W_lin, b_lin,
           ln_gamma, ln_beta):
  ids = node_ids.astype(jnp.int32)
  w_mid = W_conv[:, 0, PERIOD // 2].reshape(1, D_IN)
  old = _gather_rows(node_memories, ids)
  normed = _compute(messages, old, w_mid, W_lin,
                    b_lin.reshape(1, MEM_DIM), ln_gamma.reshape(1, MEM_DIM),
                    ln_beta.reshape(1, MEM_DIM))
  wtab = _winner(ids)
  out0 = _copy_tc(node_memories)
  tbl = jax.new_ref(out0)
  _scatter_rows(ids, wtab, normed, tbl)
  return tbl[...]


# speedup vs baseline: 1.1225x; 1.1225x over previous
import functools

import jax
import jax.numpy as jnp
from jax import lax
from jax.experimental import pallas as pl
from jax.experimental.pallas import tpu as pltpu
from jax.experimental.pallas import tpu_sc as plsc

NUM_NODES = 100000
MEM_DIM = 128
MSG_DIM = 128
PERIOD = 7
B = 16384
D_IN = MSG_DIM + MEM_DIM

NC = 2
NS = 16
L = 16
NW = NC * NS
BPW = B // NW

CP_CHUNK = 104
CP_STEPS = 32
CP_ROWS = CP_CHUNK * CP_STEPS
assert 31 * CP_ROWS >= NUM_NODES

_mesh = plsc.VectorSubcoreMesh(
    core_axis_name="c", subcore_axis_name="s", num_cores=NC, num_subcores=NS)


def _worker_id():
  return lax.axis_index("s") * NC + lax.axis_index("c")


@functools.partial(
    pl.kernel,
    out_type=jax.ShapeDtypeStruct((B, MEM_DIM), jnp.float32),
    mesh=_mesh,
    compiler_params=pltpu.CompilerParams(needs_layout_passes=False),
    scratch_types=[
        pltpu.VMEM((BPW,), jnp.int32),
        pltpu.VMEM((BPW, MEM_DIM), jnp.float32),
        pltpu.SemaphoreType.DMA,
    ],
)
def _gather_rows(table_hbm, idx_hbm, out_hbm, idx_v, rows_v, sem):
  base = _worker_id() * BPW
  pltpu.sync_copy(idx_hbm.at[pl.ds(base, BPW)], idx_v)
  pltpu.async_copy(table_hbm.at[idx_v], rows_v, sem).wait()
  pltpu.sync_copy(rows_v, out_hbm.at[pl.ds(base, BPW)])


def _compute_body(msg_ref, old_ref, wm_ref, wl_ref, bl_ref, g_ref, bt_ref,
                  out_ref):
  x1 = msg_ref[...] * wm_ref[:, :MSG_DIM]
  x2 = old_ref[...] * wm_ref[:, MSG_DIM:]
  dn = (((1,), (1,)), ((), ()))
  acc = lax.dot_general(x1, wl_ref[:, :MSG_DIM], dn,
                        preferred_element_type=jnp.float32)
  acc = acc + lax.dot_general(x2, wl_ref[:, MSG_DIM:], dn,
                              preferred_element_type=jnp.float32)
  acc = acc + bl_ref[...]
  mean = jnp.mean(acc, axis=-1, keepdims=True)
  var = jnp.mean((acc - mean) ** 2, axis=-1, keepdims=True)
  out_ref[...] = (acc - mean) / jnp.sqrt(var + 1e-5) * g_ref[...] + bt_ref[...]


_BLK = 1024
_compute = pl.pallas_call(
    _compute_body,
    grid=(B // _BLK,),
    in_specs=[
        pl.BlockSpec((_BLK, MSG_DIM), lambda i: (i, 0)),
        pl.BlockSpec((_BLK, MEM_DIM), lambda i: (i, 0)),
        pl.BlockSpec((1, D_IN), lambda i: (0, 0)),
        pl.BlockSpec((MEM_DIM, D_IN), lambda i: (0, 0)),
        pl.BlockSpec((1, MEM_DIM), lambda i: (0, 0)),
        pl.BlockSpec((1, MEM_DIM), lambda i: (0, 0)),
        pl.BlockSpec((1, MEM_DIM), lambda i: (0, 0)),
    ],
    out_specs=pl.BlockSpec((_BLK, MEM_DIM), lambda i: (i, 0)),
    out_shape=jax.ShapeDtypeStruct((B, MEM_DIM), jnp.float32),
)


@functools.partial(
    pl.kernel,
    out_type=jax.ShapeDtypeStruct((NUM_NODES,), jnp.int32),
    mesh=_mesh,
    compiler_params=pltpu.CompilerParams(needs_layout_passes=False),
    scratch_types=[
        pltpu.VMEM((NUM_NODES,), jnp.int32),
        pltpu.VMEM((B,), jnp.int32),
    ],
)
def _winner(idx_hbm, wtab_hbm, winner_v, allids):
  wid = _worker_id()
  iota = lax.iota(jnp.int32, L)

  @pl.when(wid == 0)
  def _build():
    pltpu.sync_copy(idx_hbm, allids)

    def build_outer(g):
      for j in range(16):
        o = g * 256 + j * L
        idv = allids[pl.ds(o, L)]
        _, is_last = plsc.scan_count(idv)
        plsc.store_scatter(winner_v, [idv], o + iota, mask=is_last)

    lax.fori_loop(0, B // 256, lambda g, _: (build_outer(g), 0)[1], 0)
    pltpu.sync_copy(winner_v, wtab_hbm)


def _copy_body(src_ref, dst_ref):
  dst_ref[...] = src_ref[...]


_CPBLK = 10000
_copy_tc = pl.pallas_call(
    _copy_body,
    grid=(NUM_NODES // _CPBLK,),
    in_specs=[pl.BlockSpec((_CPBLK, MEM_DIM), lambda i: (i, 0))],
    out_specs=pl.BlockSpec((_CPBLK, MEM_DIM), lambda i: (i, 0)),
    out_shape=jax.ShapeDtypeStruct((NUM_NODES, MEM_DIM), jnp.float32),
)


_SUB = 128

@functools.partial(
    pl.kernel,
    out_type=(),
    mesh=_mesh,
    compiler_params=pltpu.CompilerParams(needs_layout_passes=False),
    scratch_types=[
        pltpu.VMEM((BPW,), jnp.int32),
        pltpu.VMEM((BPW,), jnp.int32),
        pltpu.VMEM((BPW // _SUB, _SUB), jnp.int32),
        pltpu.VMEM((BPW, MEM_DIM), jnp.float32),
        pltpu.SemaphoreType.DMA,
        pltpu.SemaphoreType.DMA,
    ],
)
def _scatter_rows(idx_hbm, wtab_hbm, rows_hbm, out_hbm,
                  widxb, idsb, sidx, rbuf, sem, sem2):
  base = _worker_id() * BPW
  pltpu.sync_copy(idx_hbm.at[pl.ds(base, BPW)], idsb)
  pltpu.async_copy(wtab_hbm.at[idsb], widxb, sem).wait()
  gat = pltpu.async_copy(rows_hbm.at[widxb], rbuf, sem)
  for k in range(BPW // L):
    sidx[k * L // _SUB, pl.ds((k * L) % _SUB, L)] = idsb[pl.ds(k * L, L)]
  gat.wait()
  copies = [
      pltpu.async_copy(rbuf.at[pl.ds(s * _SUB, _SUB)],
                       out_hbm.at[sidx.at[s]], sem2)
      for s in range(BPW // _SUB)
  ]
  for c in copies:
    c.wait()


def kernel(node_ids, messages, node_memories, W_conv, W_lin, b_lin,
           ln_gamma, ln_beta):
  ids = node_ids.astype(jnp.int32)
  w_mid = W_conv[:, 0, PERIOD // 2].reshape(1, D_IN)
  old = _gather_rows(node_memories, ids)
  normed = _compute(messages, old, w_mid, W_lin,
                    b_lin.reshape(1, MEM_DIM), ln_gamma.reshape(1, MEM_DIM),
                    ln_beta.reshape(1, MEM_DIM))
  wtab = _winner(ids)
  out0 = _copy_tc(node_memories)
  tbl = jax.new_ref(out0)
  _scatter_rows(ids, wtab, normed, tbl)
  return tbl[...]


# speedup vs baseline: 1.1613x; 1.0346x over previous
import functools

import jax
import jax.numpy as jnp
from jax import lax
from jax.experimental import pallas as pl
from jax.experimental.pallas import tpu as pltpu
from jax.experimental.pallas import tpu_sc as plsc

NUM_NODES = 100000
MEM_DIM = 128
MSG_DIM = 128
PERIOD = 7
B = 16384
D_IN = MSG_DIM + MEM_DIM

NC = 2
NS = 16
L = 16
NW = NC * NS
BPW = B // NW

CP_CHUNK = 104
CP_STEPS = 32
CP_ROWS = CP_CHUNK * CP_STEPS
assert 31 * CP_ROWS >= NUM_NODES

_mesh = plsc.VectorSubcoreMesh(
    core_axis_name="c", subcore_axis_name="s", num_cores=NC, num_subcores=NS)


def _worker_id():
  return lax.axis_index("s") * NC + lax.axis_index("c")


@functools.partial(
    pl.kernel,
    out_type=jax.ShapeDtypeStruct((B, MEM_DIM), jnp.float32),
    mesh=_mesh,
    compiler_params=pltpu.CompilerParams(needs_layout_passes=False),
    scratch_types=[
        pltpu.VMEM((BPW,), jnp.int32),
        pltpu.VMEM((BPW, MEM_DIM), jnp.float32),
        pltpu.SemaphoreType.DMA,
    ],
)
def _gather_rows(table_hbm, idx_hbm, out_hbm, idx_v, rows_v, sem):
  base = _worker_id() * BPW
  pltpu.sync_copy(idx_hbm.at[pl.ds(base, BPW)], idx_v)
  pltpu.async_copy(table_hbm.at[idx_v], rows_v, sem).wait()
  pltpu.sync_copy(rows_v, out_hbm.at[pl.ds(base, BPW)])


def _compute_body(msg_ref, old_ref, wm_ref, wl_ref, bl_ref, g_ref, bt_ref,
                  out_ref):
  x1 = msg_ref[...] * wm_ref[:, :MSG_DIM]
  x2 = old_ref[...] * wm_ref[:, MSG_DIM:]
  dn = (((1,), (1,)), ((), ()))
  acc = lax.dot_general(x1, wl_ref[:, :MSG_DIM], dn,
                        preferred_element_type=jnp.float32)
  acc = acc + lax.dot_general(x2, wl_ref[:, MSG_DIM:], dn,
                              preferred_element_type=jnp.float32)
  acc = acc + bl_ref[...]
  mean = jnp.mean(acc, axis=-1, keepdims=True)
  var = jnp.mean((acc - mean) ** 2, axis=-1, keepdims=True)
  out_ref[...] = (acc - mean) / jnp.sqrt(var + 1e-5) * g_ref[...] + bt_ref[...]


_BLK = 2048
_compute = pl.pallas_call(
    _compute_body,
    grid=(B // _BLK,),
    in_specs=[
        pl.BlockSpec((_BLK, MSG_DIM), lambda i: (i, 0)),
        pl.BlockSpec((_BLK, MEM_DIM), lambda i: (i, 0)),
        pl.BlockSpec((1, D_IN), lambda i: (0, 0)),
        pl.BlockSpec((MEM_DIM, D_IN), lambda i: (0, 0)),
        pl.BlockSpec((1, MEM_DIM), lambda i: (0, 0)),
        pl.BlockSpec((1, MEM_DIM), lambda i: (0, 0)),
        pl.BlockSpec((1, MEM_DIM), lambda i: (0, 0)),
    ],
    out_specs=pl.BlockSpec((_BLK, MEM_DIM), lambda i: (i, 0)),
    out_shape=jax.ShapeDtypeStruct((B, MEM_DIM), jnp.float32),
)


@functools.partial(
    pl.kernel,
    out_type=jax.ShapeDtypeStruct((NUM_NODES,), jnp.int32),
    mesh=_mesh,
    compiler_params=pltpu.CompilerParams(needs_layout_passes=False),
    scratch_types=[
        pltpu.VMEM((NUM_NODES,), jnp.int32),
        pltpu.VMEM((B,), jnp.int32),
    ],
)
def _winner(idx_hbm, wtab_hbm, winner_v, allids):
  wid = _worker_id()
  iota = lax.iota(jnp.int32, L)

  @pl.when(wid == 0)
  def _build():
    pltpu.sync_copy(idx_hbm, allids)

    def build_outer(g):
      for j in range(16):
        o = g * 256 + j * L
        idv = allids[pl.ds(o, L)]
        _, is_last = plsc.scan_count(idv)
        plsc.store_scatter(winner_v, [idv], o + iota, mask=is_last)

    lax.fori_loop(0, B // 256, lambda g, _: (build_outer(g), 0)[1], 0)
    pltpu.sync_copy(winner_v, wtab_hbm)


def _copy_body(src_ref, dst_ref):
  dst_ref[...] = src_ref[...]


_CPBLK = 20000
_copy_tc = pl.pallas_call(
    _copy_body,
    grid=(NUM_NODES // _CPBLK,),
    in_specs=[pl.BlockSpec((_CPBLK, MEM_DIM), lambda i: (i, 0))],
    out_specs=pl.BlockSpec((_CPBLK, MEM_DIM), lambda i: (i, 0)),
    out_shape=jax.ShapeDtypeStruct((NUM_NODES, MEM_DIM), jnp.float32),
)


_SUB = 128

@functools.partial(
    pl.kernel,
    out_type=(),
    mesh=_mesh,
    compiler_params=pltpu.CompilerParams(needs_layout_passes=False),
    scratch_types=[
        pltpu.VMEM((BPW,), jnp.int32),
        pltpu.VMEM((BPW,), jnp.int32),
        pltpu.VMEM((BPW // _SUB, _SUB), jnp.int32),
        pltpu.VMEM((BPW, MEM_DIM), jnp.float32),
        pltpu.SemaphoreType.DMA,
        pltpu.SemaphoreType.DMA,
    ],
)
def _scatter_rows(idx_hbm, wtab_hbm, rows_hbm, out_hbm,
                  widxb, idsb, sidx, rbuf, sem, sem2):
  base = _worker_id() * BPW
  pltpu.sync_copy(idx_hbm.at[pl.ds(base, BPW)], idsb)
  pltpu.async_copy(wtab_hbm.at[idsb], widxb, sem).wait()
  gat = pltpu.async_copy(rows_hbm.at[widxb], rbuf, sem)
  for k in range(BPW // L):
    sidx[k * L // _SUB, pl.ds((k * L) % _SUB, L)] = idsb[pl.ds(k * L, L)]
  gat.wait()
  copies = [
      pltpu.async_copy(rbuf.at[pl.ds(s * _SUB, _SUB)],
                       out_hbm.at[sidx.at[s]], sem2)
      for s in range(BPW // _SUB)
  ]
  for c in copies:
    c.wait()


def kernel(node_ids, messages, node_memories, W_conv, W_lin, b_lin,
           ln_gamma, ln_beta):
  ids = node_ids.astype(jnp.int32)
  w_mid = W_conv[:, 0, PERIOD // 2].reshape(1, D_IN)
  old = _gather_rows(node_memories, ids)
  normed = _compute(messages, old, w_mid, W_lin,
                    b_lin.reshape(1, MEM_DIM), ln_gamma.reshape(1, MEM_DIM),
                    ln_beta.reshape(1, MEM_DIM))
  wtab = _winner(ids)
  out0 = _copy_tc(node_memories)
  tbl = jax.new_ref(out0)
  _scatter_rows(ids, wtab, normed, tbl)
  return tbl[...]


# speedup vs baseline: 1.2483x; 1.0750x over previous
import functools

import jax
import jax.numpy as jnp
from jax import lax
from jax.experimental import pallas as pl
from jax.experimental.pallas import tpu as pltpu
from jax.experimental.pallas import tpu_sc as plsc

NUM_NODES = 100000
MEM_DIM = 128
MSG_DIM = 128
PERIOD = 7
B = 16384
D_IN = MSG_DIM + MEM_DIM

NC = 2
NS = 16
L = 16
NW = NC * NS
BPW = B // NW

CP_CHUNK = 104
CP_STEPS = 32
CP_ROWS = CP_CHUNK * CP_STEPS
assert 31 * CP_ROWS >= NUM_NODES

_mesh = plsc.VectorSubcoreMesh(
    core_axis_name="c", subcore_axis_name="s", num_cores=NC, num_subcores=NS)


def _worker_id():
  return lax.axis_index("s") * NC + lax.axis_index("c")


@functools.partial(
    pl.kernel,
    out_type=(
        jax.ShapeDtypeStruct((B, MEM_DIM), jnp.float32),
        jax.ShapeDtypeStruct((NUM_NODES,), jnp.int32),
    ),
    mesh=_mesh,
    compiler_params=pltpu.CompilerParams(needs_layout_passes=False),
    scratch_types=[
        pltpu.VMEM((BPW,), jnp.int32),
        pltpu.SemaphoreType.DMA,
    ],
)
def _gather_winner(table_hbm, idx_hbm, old_hbm, wtab_hbm, idx_v, sem):
  wid = _worker_id()
  base = wid * BPW
  pltpu.sync_copy(idx_hbm.at[pl.ds(base, BPW)], idx_v)

  def _gather(rows_v):
    pltpu.async_copy(table_hbm.at[idx_v], rows_v, sem).wait()
    pltpu.sync_copy(rows_v, old_hbm.at[pl.ds(base, BPW)])

  pl.run_scoped(_gather, pltpu.VMEM((BPW, MEM_DIM), jnp.float32))

  @pl.when(wid == 0)
  def _build():
    iota = lax.iota(jnp.int32, L)

    def _w(winner_v, allids):
      pltpu.sync_copy(idx_hbm, allids)

      def build_outer(g):
        for j in range(16):
          o = g * 256 + j * L
          idv = allids[pl.ds(o, L)]
          _, is_last = plsc.scan_count(idv)
          plsc.store_scatter(winner_v, [idv], o + iota, mask=is_last)

      lax.fori_loop(0, B // 256, lambda g, _: (build_outer(g), 0)[1], 0)
      pltpu.sync_copy(winner_v, wtab_hbm)

    pl.run_scoped(_w, pltpu.VMEM((NUM_NODES,), jnp.int32),
                  pltpu.VMEM((B,), jnp.int32))


def _compute_body(msg_ref, old_ref, wm_ref, wl_ref, bl_ref, g_ref, bt_ref,
                  out_ref):
  x1 = msg_ref[...] * wm_ref[:, :MSG_DIM]
  x2 = old_ref[...] * wm_ref[:, MSG_DIM:]
  dn = (((1,), (1,)), ((), ()))
  acc = lax.dot_general(x1, wl_ref[:, :MSG_DIM], dn,
                        preferred_element_type=jnp.float32)
  acc = acc + lax.dot_general(x2, wl_ref[:, MSG_DIM:], dn,
                              preferred_element_type=jnp.float32)
  acc = acc + bl_ref[...]
  mean = jnp.mean(acc, axis=-1, keepdims=True)
  var = jnp.mean((acc - mean) ** 2, axis=-1, keepdims=True)
  out_ref[...] = (acc - mean) / jnp.sqrt(var + 1e-5) * g_ref[...] + bt_ref[...]


_BLK = 4096
_compute = pl.pallas_call(
    _compute_body,
    grid=(B // _BLK,),
    in_specs=[
        pl.BlockSpec((_BLK, MSG_DIM), lambda i: (i, 0)),
        pl.BlockSpec((_BLK, MEM_DIM), lambda i: (i, 0)),
        pl.BlockSpec((1, D_IN), lambda i: (0, 0)),
        pl.BlockSpec((MEM_DIM, D_IN), lambda i: (0, 0)),
        pl.BlockSpec((1, MEM_DIM), lambda i: (0, 0)),
        pl.BlockSpec((1, MEM_DIM), lambda i: (0, 0)),
        pl.BlockSpec((1, MEM_DIM), lambda i: (0, 0)),
    ],
    out_specs=pl.BlockSpec((_BLK, MEM_DIM), lambda i: (i, 0)),
    out_shape=jax.ShapeDtypeStruct((B, MEM_DIM), jnp.float32),
)


def _copy_body(src_ref, dst_ref):
  dst_ref[...] = src_ref[...]


_CPBLK = 20000
_copy_tc = pl.pallas_call(
    _copy_body,
    grid=(NUM_NODES // _CPBLK,),
    in_specs=[pl.BlockSpec((_CPBLK, MEM_DIM), lambda i: (i, 0))],
    out_specs=pl.BlockSpec((_CPBLK, MEM_DIM), lambda i: (i, 0)),
    out_shape=jax.ShapeDtypeStruct((NUM_NODES, MEM_DIM), jnp.float32),
)


_SUB = 128

@functools.partial(
    pl.kernel,
    out_type=(),
    mesh=_mesh,
    compiler_params=pltpu.CompilerParams(needs_layout_passes=False),
    scratch_types=[
        pltpu.VMEM((BPW,), jnp.int32),
        pltpu.VMEM((BPW,), jnp.int32),
        pltpu.VMEM((BPW // _SUB, _SUB), jnp.int32),
        pltpu.VMEM((BPW, MEM_DIM), jnp.float32),
        pltpu.SemaphoreType.DMA,
        pltpu.SemaphoreType.DMA,
    ],
)
def _scatter_rows(idx_hbm, wtab_hbm, rows_hbm, out_hbm,
                  widxb, idsb, sidx, rbuf, sem, sem2):
  base = _worker_id() * BPW
  pltpu.sync_copy(idx_hbm.at[pl.ds(base, BPW)], idsb)
  pltpu.async_copy(wtab_hbm.at[idsb], widxb, sem).wait()
  gat = pltpu.async_copy(rows_hbm.at[widxb], rbuf, sem)
  for k in range(BPW // L):
    sidx[k * L // _SUB, pl.ds((k * L) % _SUB, L)] = idsb[pl.ds(k * L, L)]
  gat.wait()
  copies = [
      pltpu.async_copy(rbuf.at[pl.ds(s * _SUB, _SUB)],
                       out_hbm.at[sidx.at[s]], sem2)
      for s in range(BPW // _SUB)
  ]
  for c in copies:
    c.wait()


def kernel(node_ids, messages, node_memories, W_conv, W_lin, b_lin,
           ln_gamma, ln_beta):
  ids = node_ids.astype(jnp.int32)
  w_mid = W_conv[:, 0, PERIOD // 2].reshape(1, D_IN)
  old, wtab = _gather_winner(node_memories, ids)
  normed = _compute(messages, old, w_mid, W_lin,
                    b_lin.reshape(1, MEM_DIM), ln_gamma.reshape(1, MEM_DIM),
                    ln_beta.reshape(1, MEM_DIM))
  out0 = _copy_tc(node_memories)
  tbl = jax.new_ref(out0)
  _scatter_rows(ids, wtab, normed, tbl)
  return tbl[...]
